# baseline (speedup 1.0000x reference)
import functools

import jax
import jax.numpy as jnp
from jax import lax
from jax.experimental import pallas as pl
from jax.experimental.pallas import tpu as pltpu
from jax.experimental.pallas import tpu_sc as plsc

NUM_EMB = 1000000
DIM = 64
N_TOKENS = 4096 * 200


@functools.lru_cache(maxsize=None)
def _make_lookup(n, v, d):
    info = plsc.get_sparse_core_info()
    nw = info.num_cores * info.num_subcores
    n_per_w = n // nw
    chunk = 1024
    n_chunks = n_per_w // chunk

    mesh = plsc.VectorSubcoreMesh(core_axis_name="c", subcore_axis_name="s")

    @functools.partial(
        pl.kernel,
        mesh=mesh,
        out_type=jax.ShapeDtypeStruct((n, d), jnp.float32),
        compiler_params=pltpu.CompilerParams(use_tc_tiling_on_sc=False),
        scratch_types=[
            pltpu.VMEM((chunk,), jnp.int32),
            pltpu.VMEM((chunk, d), jnp.float32),
            pltpu.SemaphoreType.DMA,
        ],
    )
    def lookup(idx_hbm, table_hbm, out_hbm, idx_v, rows_v, sem):
        wid = lax.axis_index("s") * info.num_cores + lax.axis_index("c")
        base = wid * n_per_w

        def body(i, carry):
            off = base + i * chunk
            pltpu.sync_copy(idx_hbm.at[pl.ds(off, chunk)], idx_v)
            pltpu.async_copy(table_hbm.at[idx_v], rows_v, sem).wait()
            pltpu.sync_copy(rows_v, out_hbm.at[pl.ds(off, chunk)])
            return carry

        lax.fori_loop(0, n_chunks, body, 0)

    return lookup


def kernel(x, embedding):
    b, t = x.shape
    flat_x = x.reshape(-1).astype(jnp.int32)
    out = _make_lookup(b * t, embedding.shape[0], embedding.shape[1])(
        flat_x, embedding)
    return out.reshape(b, t, embedding.shape[1])


# speedup vs baseline: 1.0134x; 1.0134x over previous
import functools

import jax
import jax.numpy as jnp
from jax import lax
from jax.experimental import pallas as pl
from jax.experimental.pallas import tpu as pltpu
from jax.experimental.pallas import tpu_sc as plsc


@functools.lru_cache(maxsize=None)
def _make_lookup(n, v, d):
    info = plsc.get_sparse_core_info()
    nw = info.num_cores * info.num_subcores
    n_per_w = n // nw
    chunk = 256
    nbuf = 4
    n_chunks = n_per_w // chunk
    n_outer = n_chunks // nbuf

    mesh = plsc.VectorSubcoreMesh(core_axis_name="c", subcore_axis_name="s")

    @functools.partial(
        pl.kernel,
        mesh=mesh,
        out_type=jax.ShapeDtypeStruct((n, d), jnp.float32),
        compiler_params=pltpu.CompilerParams(use_tc_tiling_on_sc=False),
        scratch_types=[
            pltpu.VMEM((n_per_w,), jnp.int32),
            pltpu.VMEM((nbuf, chunk, d), jnp.float32),
            pltpu.SemaphoreType.DMA,
            pltpu.SemaphoreType.DMA,
        ],
    )
    def lookup(idx_hbm, table_hbm, out_hbm, idx_v, rows_v, gsem, wsem):
        wid = lax.axis_index("s") * info.num_cores + lax.axis_index("c")
        base = wid * n_per_w

        pltpu.sync_copy(idx_hbm.at[pl.ds(base, n_per_w)], idx_v)

        def gather_copy(g, b):
            return pltpu.make_async_copy(
                table_hbm.at[idx_v.at[pl.ds(g * chunk, chunk)]],
                rows_v.at[b], gsem)

        def wb_copy(g, b):
            return pltpu.make_async_copy(
                rows_v.at[b], out_hbm.at[pl.ds(base + g * chunk, chunk)],
                wsem)

        for b in range(nbuf):
            gather_copy(b, b).start()

        def outer(k, carry):
            for b in range(nbuf):
                g = k * nbuf + b
                gather_copy(g, b).wait()
                wb_copy(g, b).start()
                wb_copy(g, b).wait()

                @pl.when(g + nbuf < n_chunks)
                def _():
                    gather_copy(g + nbuf, b).start()
            return carry

        lax.fori_loop(0, n_outer, outer, 0)

    return lookup


def kernel(x, embedding):
    b, t = x.shape
    flat_x = x.reshape(-1).astype(jnp.int32)
    out = _make_lookup(b * t, embedding.shape[0], embedding.shape[1])(
        flat_x, embedding)
    return out.reshape(b, t, embedding.shape[1])


# speedup vs baseline: 1.0159x; 1.0024x over previous
import functools

import jax
import jax.numpy as jnp
from jax import lax
from jax.experimental import pallas as pl
from jax.experimental.pallas import tpu as pltpu
from jax.experimental.pallas import tpu_sc as plsc


@functools.lru_cache(maxsize=None)
def _make_lookup(n, v, d):
    info = plsc.get_sparse_core_info()
    nw = info.num_cores * info.num_subcores
    n_per_w = n // nw
    chunk = 800
    nbuf = 2
    n_chunks = n_per_w // chunk
    n_outer = n_chunks // nbuf

    mesh = plsc.VectorSubcoreMesh(core_axis_name="c", subcore_axis_name="s")

    @functools.partial(
        pl.kernel,
        mesh=mesh,
        out_type=jax.ShapeDtypeStruct((n, d), jnp.float32),
        compiler_params=pltpu.CompilerParams(use_tc_tiling_on_sc=False),
        scratch_types=[
            pltpu.VMEM((n_per_w,), jnp.int32),
            pltpu.VMEM((nbuf, chunk, d), jnp.float32),
            pltpu.SemaphoreType.DMA,
            pltpu.SemaphoreType.DMA,
        ],
    )
    def lookup(idx_hbm, table_hbm, out_hbm, idx_v, rows_v, gsem, wsem):
        wid = lax.axis_index("s") * info.num_cores + lax.axis_index("c")
        base = wid * n_per_w

        pltpu.sync_copy(idx_hbm.at[pl.ds(base, n_per_w)], idx_v)

        def gather_copy(g, b):
            return pltpu.make_async_copy(
                table_hbm.at[idx_v.at[pl.ds(g * chunk, chunk)]],
                rows_v.at[b], gsem)

        def wb_copy(g, b):
            return pltpu.make_async_copy(
                rows_v.at[b], out_hbm.at[pl.ds(base + g * chunk, chunk)],
                wsem)

        for b in range(nbuf):
            gather_copy(b, b).start()

        def outer(k, carry):
            for b in range(nbuf):
                g = k * nbuf + b
                gather_copy(g, b).wait()
                wb_copy(g, b).start()
                wb_copy(g, b).wait()

                @pl.when(g + nbuf < n_chunks)
                def _():
                    gather_copy(g + nbuf, b).start()
            return carry

        lax.fori_loop(0, n_outer, outer, 0)

    return lookup


def kernel(x, embedding):
    b, t = x.shape
    flat_x = x.reshape(-1).astype(jnp.int32)
    out = _make_lookup(b * t, embedding.shape[0], embedding.shape[1])(
        flat_x, embedding)
    return out.reshape(b, t, embedding.shape[1])
